# baseline (speedup 1.0000x reference)
import functools

import jax
import jax.numpy as jnp
from jax.experimental import pallas as pl
from jax.experimental.pallas import tpu as pltpu


def _gcn_kernel(adj_ref, x_ref, w_ref, b_ref, out_ref, *, blk, n):
    pid = pl.program_id(0)
    a = adj_ref[:, :]
    xfull = x_ref[pl.ds(0, n), :]

    ax = jnp.dot(a, xfull, preferred_element_type=jnp.float32)
    w = w_ref[:, :]
    s = jnp.dot(ax, w, preferred_element_type=jnp.float32)

    deg = jnp.sum(a, axis=1, keepdims=True)

    win_max = (n // 128) * 128 - blk + 128
    start = pl.multiple_of(jnp.minimum(pid * blk, win_max), 128)
    off = pid * blk - start
    dsub = adj_ref[:, pl.ds(start, blk)]
    rows = jax.lax.broadcasted_iota(jnp.int32, (blk, blk), 0)
    cols = jax.lax.broadcasted_iota(jnp.int32, (blk, blk), 1)
    d = jnp.sum(jnp.where(cols == rows + off, dsub, 0.0), axis=1,
                keepdims=True)

    xb = x_ref[pl.ds(pid * blk, blk), :]
    xw = jnp.dot(xb, w, preferred_element_type=jnp.float32)

    inv = jnp.where(deg > 0.0, 1.0 / deg, 0.0)
    out_ref[:, :] = xw - inv * (2.0 * s * xw - 2.0 * d * xw * xw) - b_ref[:, :]


def kernel(x, adj, edge_weight, W, b):
    del edge_weight
    n, d_in = x.shape
    d_out = W.shape[1]
    blk = 512
    grid = (pl.cdiv(n, blk),)
    n_pad = grid[0] * blk
    xp = jnp.pad(x, ((0, n_pad - n), (0, 0)))
    out = pl.pallas_call(
        functools.partial(_gcn_kernel, blk=blk, n=n),
        grid=grid,
        in_specs=[
            pl.BlockSpec((blk, n), lambda i: (i, 0)),
            pl.BlockSpec((n_pad, d_in), lambda i: (0, 0)),
            pl.BlockSpec((d_in, d_out), lambda i: (0, 0)),
            pl.BlockSpec((1, d_out), lambda i: (0, 0)),
        ],
        out_specs=pl.BlockSpec((blk, d_out), lambda i: (i, 0)),
        out_shape=jax.ShapeDtypeStruct((n, d_out), jnp.float32),
        compiler_params=pltpu.CompilerParams(
            dimension_semantics=("parallel",)),
    )(adj, xp, W, b.reshape(1, d_out))
    return out


# speedup vs baseline: 1.0358x; 1.0358x over previous
import functools

import jax
import jax.numpy as jnp
from jax.experimental import pallas as pl
from jax.experimental.pallas import tpu as pltpu


def _gcn_kernel(adj_ref, x_ref, xb_ref, w_ref, b_ref, out_ref, *, blk, n):
    pid = pl.program_id(0)
    a = adj_ref[:, :]
    xfull = x_ref[:, :]

    ax = jnp.dot(a, xfull, preferred_element_type=jnp.float32)
    w = w_ref[:, :]
    s = jnp.dot(ax, w, preferred_element_type=jnp.float32)

    deg = jnp.sum(a, axis=1, keepdims=True)

    win_max = (n // 128) * 128 - blk + 128
    start = pl.multiple_of(jnp.minimum(pid * blk, win_max), 128)
    off = pid * blk - start
    dsub = adj_ref[:, pl.ds(start, blk)]
    rows = jax.lax.broadcasted_iota(jnp.int32, (blk, blk), 0)
    cols = jax.lax.broadcasted_iota(jnp.int32, (blk, blk), 1)
    d = jnp.sum(jnp.where(cols == rows + off, dsub, 0.0), axis=1,
                keepdims=True)

    xw = jnp.dot(xb_ref[:, :], w, preferred_element_type=jnp.float32)

    inv = jnp.where(deg > 0.0, 1.0 / deg, 0.0)
    out_ref[:, :] = xw - inv * (2.0 * s * xw - 2.0 * d * xw * xw) - b_ref[:, :]


def kernel(x, adj, edge_weight, W, b):
    del edge_weight
    n, d_in = x.shape
    d_out = W.shape[1]
    blk = 256
    grid = (pl.cdiv(n, blk),)
    out = pl.pallas_call(
        functools.partial(_gcn_kernel, blk=blk, n=n),
        grid=grid,
        in_specs=[
            pl.BlockSpec((blk, n), lambda i: (i, 0)),
            pl.BlockSpec((n, d_in), lambda i: (0, 0)),
            pl.BlockSpec((blk, d_in), lambda i: (i, 0)),
            pl.BlockSpec((d_in, d_out), lambda i: (0, 0)),
            pl.BlockSpec((1, d_out), lambda i: (0, 0)),
        ],
        out_specs=pl.BlockSpec((blk, d_out), lambda i: (i, 0)),
        out_shape=jax.ShapeDtypeStruct((n, d_out), jnp.float32),
        compiler_params=pltpu.CompilerParams(
            dimension_semantics=("parallel",)),
    )(adj, x, x, W, b.reshape(1, d_out))
    return out
